# baseline (speedup 1.0000x reference)
import jax
import jax.numpy as jnp
from jax import lax
from jax.experimental import pallas as pl
from jax.experimental.pallas import tpu as pltpu
from jax.experimental.pallas import tpu_sc as plsc

VOCAB = 1_000_000
MOD = VOCAB - 1
BIGRAM_DIM = 64
MODEL_DIM = 1024
BATCH = 4
SEQ = 4096
TOKENS = BATCH * SEQ
NW = 32
PAD = 16
LANES = 16
GSZ = 4
NCHUNK = 2
CHUNK = TOKENS // NCHUNK
PER_W = CHUNK // NW
NVEC = PER_W // LANES
NSR = PER_W // GSZ
NPIPE = (NSR - 4) // 12
TOK_BLK = 1024


def _iota16():
    return lax.iota(jnp.int32, LANES)


def _splat(x):
    return jnp.full((LANES,), x, jnp.int32)


def _make_sc_body(chunk_off):
    def _sc_body(tok_hbm, tabT_hbm, outT_hbm, tok_v, idx_v, tc_v, col_v,
                 sem_a, sem_b, sem_c):
        wid = lax.axis_index("s") * 2 + lax.axis_index("c")
        base = wid * PER_W
        gbase = chunk_off + base
        pltpu.sync_copy(tok_hbm.at[pl.ds(gbase, PAD + PER_W)], tok_v)

        for v in range(NVEC):
            curr = tok_v[pl.ds(PAD + v * LANES, LANES)]
            prev = tok_v[pl.ds(PAD - 1 + v * LANES, LANES)]
            a = jnp.int32(36313) * curr
            b = jnp.int32(27191) * prev
            h = lax.rem(a ^ b, jnp.int32(MOD))
            h = jnp.where(h < 0, h + jnp.int32(MOD), h)
            pos = gbase + v * LANES + _iota16()
            h = jnp.where((pos & jnp.int32(SEQ - 1)) == 0, jnp.int32(MOD), h)
            idx_v[pl.ds(v * LANES, LANES)] = h

        sems = (sem_a, sem_b, sem_c)

        def fire(cvec, s, q, sem):
            for k in range(GSZ):
                c = cvec[GSZ * s + k]
                off = pl.multiple_of(c * jnp.int32(128), 128)
                pltpu.make_async_copy(
                    tabT_hbm.at[:, pl.ds(off, 128)],
                    tc_v.at[pl.ds((q * GSZ + k) * BIGRAM_DIM, BIGRAM_DIM)],
                    sem,
                ).start()

        def drain(q, sem):
            for k in range(GSZ):
                pltpu.make_async_copy(
                    tabT_hbm.at[:, pl.ds(0, 128)],
                    tc_v.at[pl.ds((q * GSZ + k) * BIGRAM_DIM, BIGRAM_DIM)],
                    sem,
                ).wait()

        def extract(lvec, tpos0, s, q):
            for k in range(GSZ):
                lane = lvec[GSZ * s + k]
                tpos = tpos0 + jnp.int32(GSZ * s + k)
                row0 = (q * GSZ + k) * BIGRAM_DIM
                for k2 in range(BIGRAM_DIM // LANES):
                    vec = plsc.load_gather(
                        tc_v,
                        [_splat(row0 + k2 * LANES) + _iota16(), _splat(lane)],
                    )
                    plsc.store_scatter(
                        col_v, [_splat(k2 * LANES) + _iota16(), _splat(tpos)],
                        vec,
                    )

        def flush(blk):
            off = pl.multiple_of(base + blk * jnp.int32(128), 128)
            pltpu.sync_copy(col_v, outT_hbm.at[:, pl.ds(off, 128)])

        def group_vecs(g):
            vec = idx_v[pl.ds(g * LANES, LANES)]
            return (lax.shift_right_logical(vec, 7), vec & jnp.int32(127),
                    (g & jnp.int32(7)) * LANES)

        cv0, lv0, tp0 = group_vecs(jnp.int32(0))
        fire(cv0, 0, 0, sems[0])
        fire(cv0, 1, 1, sems[1])

        def pipe(i, carry):
            gv = [group_vecs(jnp.int32(3) * i + jnp.int32(m))
                  for m in range(4)]
            for j in range(12):
                sr_f = 2 + j
                fire(gv[sr_f // 4][0], sr_f % 4, sr_f % 3, sems[sr_f % 3])
                sr_d = j
                drain(sr_d % 3, sems[sr_d % 3])
                extract(gv[sr_d // 4][1], gv[sr_d // 4][2], sr_d % 4,
                        sr_d % 3)

                @pl.when(((jnp.int32(12) * i + jnp.int32(sr_d))
                          & jnp.int32(31)) == jnp.int32(31))
                def _():
                    flush(lax.shift_right_logical(
                        jnp.int32(12) * i + jnp.int32(sr_d), 5))

            return carry

        lax.fori_loop(0, NPIPE, pipe, 0)

        gvs = {}

        def gv_of(sr):
            g = sr // 4
            if g not in gvs:
                gvs[g] = group_vecs(jnp.int32(g))
            return gvs[g]

        for t in range(12 * NPIPE, NSR):
            if t + 2 < NSR:
                fire(gv_of(t + 2)[0], (t + 2) % 4, (t + 2) % 3,
                     sems[(t + 2) % 3])
            drain(t % 3, sems[t % 3])
            extract(gv_of(t)[1], gv_of(t)[2], t % 4, t % 3)
            if (t & 31) == 31:
                flush(jnp.int32(t >> 5))

    return _sc_body


def _sc_hash_gather(tok_padded, tableT, chunk):
    mesh = plsc.VectorSubcoreMesh(
        core_axis_name="c", subcore_axis_name="s", num_cores=2, num_subcores=16
    )
    return pl.kernel(
        _make_sc_body(chunk * CHUNK),
        out_type=jax.ShapeDtypeStruct((BIGRAM_DIM, CHUNK), jnp.float32),
        mesh=mesh,
        scratch_types=[
            pltpu.VMEM((PAD + PER_W,), jnp.int32),
            pltpu.VMEM((PER_W,), jnp.int32),
            pltpu.VMEM((3 * GSZ * BIGRAM_DIM, 128), jnp.float32),
            pltpu.VMEM((BIGRAM_DIM, 128), jnp.float32),
            pltpu.SemaphoreType.DMA,
            pltpu.SemaphoreType.DMA,
            pltpu.SemaphoreType.DMA,
        ],
        compiler_params=pltpu.CompilerParams(needs_layout_passes=False),
    )(tok_padded, tableT)


def _tc_proj_body0(scale_ref, gT_ref, p_ref, o_ref):
    acc = lax.dot_general(
        gT_ref[...], p_ref[...], (((0,), (1,)), ((), ())),
        preferred_element_type=jnp.float32,
    )
    o_ref[...] = acc * scale_ref[0, 0]


def _tc_proj_body1(scale_ref, gT_ref, p_ref, prev_ref, o_ref):
    _tc_proj_body0(scale_ref, gT_ref, p_ref, o_ref)


def _tc_proj_chunk(gatheredT, proj, scale, prev, chunk):
    blk0 = chunk * (CHUNK // TOK_BLK)
    common = dict(
        grid=(CHUNK // TOK_BLK,),
        out_specs=pl.BlockSpec((TOK_BLK, MODEL_DIM),
                               lambda i, b=blk0: (i + b, 0)),
        out_shape=jax.ShapeDtypeStruct((TOKENS, MODEL_DIM), jnp.float32),
    )
    in_specs = [
        pl.BlockSpec(memory_space=pltpu.SMEM),
        pl.BlockSpec((BIGRAM_DIM, TOK_BLK), lambda i: (0, i)),
        pl.BlockSpec((MODEL_DIM, BIGRAM_DIM), lambda i: (0, 0)),
    ]
    if prev is None:
        return pl.pallas_call(
            _tc_proj_body0, in_specs=in_specs, **common,
        )(scale.reshape(1, 1), gatheredT, proj)
    return pl.pallas_call(
        _tc_proj_body1,
        in_specs=in_specs + [pl.BlockSpec(memory_space=pl.ANY)],
        input_output_aliases={3: 0},
        **common,
    )(scale.reshape(1, 1), gatheredT, proj, prev)


def kernel(tokens, embed_weight, proj_weight, scale):
    tok_flat = tokens.astype(jnp.int32).reshape(-1)
    tok_padded = jnp.concatenate([jnp.zeros((PAD,), jnp.int32), tok_flat])
    tabT = embed_weight.T
    gT = [_sc_hash_gather(tok_padded, tabT, c) for c in range(NCHUNK)]
    out = None
    for c in range(NCHUNK):
        out = _tc_proj_chunk(gT[c], proj_weight, scale, out, c)
    return out.reshape(BATCH, SEQ, MODEL_DIM)


# speedup vs baseline: 1.0042x; 1.0042x over previous
import jax
import jax.numpy as jnp
from jax import lax
from jax.experimental import pallas as pl
from jax.experimental.pallas import tpu as pltpu
from jax.experimental.pallas import tpu_sc as plsc

VOCAB = 1_000_000
MOD = VOCAB - 1
BIGRAM_DIM = 64
MODEL_DIM = 1024
BATCH = 4
SEQ = 4096
TOKENS = BATCH * SEQ
NW = 32
PAD = 16
LANES = 16
GSZ = 4
NCHUNK = 1
CHUNK = TOKENS // NCHUNK
PER_W = CHUNK // NW
NVEC = PER_W // LANES
NSR = PER_W // GSZ
NPIPE = (NSR - 4) // 12
TOK_BLK = 1024


def _iota16():
    return lax.iota(jnp.int32, LANES)


def _splat(x):
    return jnp.full((LANES,), x, jnp.int32)


def _make_sc_body(chunk_off):
    def _sc_body(tok_hbm, tabT_hbm, outT_hbm, tok_v, idx_v, tc_v, col_v,
                 sem_a, sem_b, sem_c):
        wid = lax.axis_index("s") * 2 + lax.axis_index("c")
        base = wid * PER_W
        gbase = chunk_off + base
        pltpu.sync_copy(tok_hbm.at[pl.ds(gbase, PAD + PER_W)], tok_v)

        for v in range(NVEC):
            curr = tok_v[pl.ds(PAD + v * LANES, LANES)]
            prev = tok_v[pl.ds(PAD - 1 + v * LANES, LANES)]
            a = jnp.int32(36313) * curr
            b = jnp.int32(27191) * prev
            h = lax.rem(a ^ b, jnp.int32(MOD))
            h = jnp.where(h < 0, h + jnp.int32(MOD), h)
            pos = gbase + v * LANES + _iota16()
            h = jnp.where((pos & jnp.int32(SEQ - 1)) == 0, jnp.int32(MOD), h)
            idx_v[pl.ds(v * LANES, LANES)] = h

        sems = (sem_a, sem_b, sem_c)

        def fire(cvec, s, q, sem):
            for k in range(GSZ):
                c = cvec[GSZ * s + k]
                off = pl.multiple_of(c * jnp.int32(128), 128)
                pltpu.make_async_copy(
                    tabT_hbm.at[:, pl.ds(off, 128)],
                    tc_v.at[pl.ds((q * GSZ + k) * BIGRAM_DIM, BIGRAM_DIM)],
                    sem,
                ).start()

        def drain(q, sem):
            for k in range(GSZ):
                pltpu.make_async_copy(
                    tabT_hbm.at[:, pl.ds(0, 128)],
                    tc_v.at[pl.ds((q * GSZ + k) * BIGRAM_DIM, BIGRAM_DIM)],
                    sem,
                ).wait()

        def extract(lvec, tpos0, s, q):
            for k in range(GSZ):
                lane = lvec[GSZ * s + k]
                tpos = tpos0 + jnp.int32(GSZ * s + k)
                row0 = (q * GSZ + k) * BIGRAM_DIM
                for k2 in range(BIGRAM_DIM // LANES):
                    vec = plsc.load_gather(
                        tc_v,
                        [_splat(row0 + k2 * LANES) + _iota16(), _splat(lane)],
                    )
                    plsc.store_scatter(
                        col_v, [_splat(tpos), _splat(k2 * LANES) + _iota16()],
                        vec,
                    )

        def flush(blk):
            off = pl.multiple_of(base + blk * jnp.int32(128), 128)
            pltpu.sync_copy(col_v, outT_hbm.at[pl.ds(off, 128)])

        def group_vecs(g):
            vec = idx_v[pl.ds(g * LANES, LANES)]
            return (lax.shift_right_logical(vec, 7), vec & jnp.int32(127),
                    (g & jnp.int32(7)) * LANES)

        cv0, lv0, tp0 = group_vecs(jnp.int32(0))
        fire(cv0, 0, 0, sems[0])
        fire(cv0, 1, 1, sems[1])

        def pipe(i, carry):
            gv = [group_vecs(jnp.int32(3) * i + jnp.int32(m))
                  for m in range(4)]
            for j in range(12):
                sr_f = 2 + j
                fire(gv[sr_f // 4][0], sr_f % 4, sr_f % 3, sems[sr_f % 3])
                sr_d = j
                drain(sr_d % 3, sems[sr_d % 3])
                extract(gv[sr_d // 4][1], gv[sr_d // 4][2], sr_d % 4,
                        sr_d % 3)

                @pl.when(((jnp.int32(12) * i + jnp.int32(sr_d))
                          & jnp.int32(31)) == jnp.int32(31))
                def _():
                    flush(lax.shift_right_logical(
                        jnp.int32(12) * i + jnp.int32(sr_d), 5))

            return carry

        lax.fori_loop(0, NPIPE, pipe, 0)

        gvs = {}

        def gv_of(sr):
            g = sr // 4
            if g not in gvs:
                gvs[g] = group_vecs(jnp.int32(g))
            return gvs[g]

        for t in range(12 * NPIPE, NSR):
            if t + 2 < NSR:
                fire(gv_of(t + 2)[0], (t + 2) % 4, (t + 2) % 3,
                     sems[(t + 2) % 3])
            drain(t % 3, sems[t % 3])
            extract(gv_of(t)[1], gv_of(t)[2], t % 4, t % 3)
            if (t & 31) == 31:
                flush(jnp.int32(t >> 5))

    return _sc_body


def _sc_hash_gather(tok_padded, tableT, chunk):
    mesh = plsc.VectorSubcoreMesh(
        core_axis_name="c", subcore_axis_name="s", num_cores=2, num_subcores=16
    )
    return pl.kernel(
        _make_sc_body(chunk * CHUNK),
        out_type=jax.ShapeDtypeStruct((CHUNK, BIGRAM_DIM), jnp.float32),
        mesh=mesh,
        scratch_types=[
            pltpu.VMEM((PAD + PER_W,), jnp.int32),
            pltpu.VMEM((PER_W,), jnp.int32),
            pltpu.VMEM((3 * GSZ * BIGRAM_DIM, 128), jnp.float32),
            pltpu.VMEM((128, BIGRAM_DIM), jnp.float32),
            pltpu.SemaphoreType.DMA,
            pltpu.SemaphoreType.DMA,
            pltpu.SemaphoreType.DMA,
        ],
        compiler_params=pltpu.CompilerParams(needs_layout_passes=False),
    )(tok_padded, tableT)


def _tc_proj_body0(scale_ref, g_ref, p_ref, o_ref):
    acc = lax.dot_general(
        g_ref[...], p_ref[...], (((1,), (1,)), ((), ())),
        preferred_element_type=jnp.float32,
    )
    o_ref[...] = acc * scale_ref[0, 0]


def _tc_proj_body1(scale_ref, gT_ref, p_ref, prev_ref, o_ref):
    _tc_proj_body0(scale_ref, gT_ref, p_ref, o_ref)


def _tc_proj_chunk(gatheredT, proj, scale, prev, chunk):
    blk0 = chunk * (CHUNK // TOK_BLK)
    common = dict(
        grid=(CHUNK // TOK_BLK,),
        out_specs=pl.BlockSpec((TOK_BLK, MODEL_DIM),
                               lambda i, b=blk0: (i + b, 0)),
        out_shape=jax.ShapeDtypeStruct((TOKENS, MODEL_DIM), jnp.float32),
    )
    in_specs = [
        pl.BlockSpec(memory_space=pltpu.SMEM),
        pl.BlockSpec((TOK_BLK, BIGRAM_DIM), lambda i: (i, 0)),
        pl.BlockSpec((MODEL_DIM, BIGRAM_DIM), lambda i: (0, 0)),
    ]
    if prev is None:
        return pl.pallas_call(
            _tc_proj_body0, in_specs=in_specs, **common,
        )(scale.reshape(1, 1), gatheredT, proj)
    return pl.pallas_call(
        _tc_proj_body1,
        in_specs=in_specs + [pl.BlockSpec(memory_space=pl.ANY)],
        input_output_aliases={3: 0},
        **common,
    )(scale.reshape(1, 1), gatheredT, proj, prev)


def kernel(tokens, embed_weight, proj_weight, scale):
    tok_flat = tokens.astype(jnp.int32).reshape(-1)
    tok_padded = jnp.concatenate([jnp.zeros((PAD,), jnp.int32), tok_flat])
    tabT = embed_weight.T
    gT = [_sc_hash_gather(tok_padded, tabT, c) for c in range(NCHUNK)]
    out = None
    for c in range(NCHUNK):
        out = _tc_proj_chunk(gT[c], proj_weight, scale, out, c)
    return out.reshape(BATCH, SEQ, MODEL_DIM)


# speedup vs baseline: 1.0066x; 1.0024x over previous
import jax
import jax.numpy as jnp
from jax import lax
from jax.experimental import pallas as pl
from jax.experimental.pallas import tpu as pltpu
from jax.experimental.pallas import tpu_sc as plsc

VOCAB = 1_000_000
MOD = VOCAB - 1
BIGRAM_DIM = 64
MODEL_DIM = 1024
BATCH = 4
SEQ = 4096
TOKENS = BATCH * SEQ
NW = 32
PAD = 16
LANES = 16
GSZ = 4
NCHUNK = 1
CHUNK = TOKENS // NCHUNK
PER_W = CHUNK // NW
NVEC = PER_W // LANES
NSR = PER_W // GSZ
NPIPE = (NSR - 4) // 12
TOK_BLK = 1024


def _iota16():
    return lax.iota(jnp.int32, LANES)


def _splat(x):
    return jnp.full((LANES,), x, jnp.int32)


def _make_sc_body(chunk_off):
    def _sc_body(tok_hbm, tabT_hbm, outT_hbm, tok_v, idx_v, tc_v, col_v,
                 sem_a, sem_b, sem_c):
        wid = lax.axis_index("s") * 2 + lax.axis_index("c")
        base = wid * PER_W
        gbase = chunk_off + base
        pltpu.sync_copy(tok_hbm.at[pl.ds(gbase, PAD + PER_W)], tok_v)

        for v in range(NVEC):
            curr = tok_v[pl.ds(PAD + v * LANES, LANES)]
            prev = tok_v[pl.ds(PAD - 1 + v * LANES, LANES)]
            a = jnp.int32(36313) * curr
            b = jnp.int32(27191) * prev
            h = lax.rem(a ^ b, jnp.int32(MOD))
            h = jnp.where(h < 0, h + jnp.int32(MOD), h)
            pos = gbase + v * LANES + _iota16()
            h = jnp.where((pos & jnp.int32(SEQ - 1)) == 0, jnp.int32(MOD), h)
            idx_v[pl.ds(v * LANES, LANES)] = h

        sems = (sem_a, sem_b, sem_c)

        def fire(cvec, s, q, sem):
            for k in range(GSZ):
                c = cvec[GSZ * s + k]
                off = pl.multiple_of(c * jnp.int32(128), 128)
                pltpu.make_async_copy(
                    tabT_hbm.at[:, pl.ds(off, 128)],
                    tc_v.at[pl.ds((q * GSZ + k) * BIGRAM_DIM, BIGRAM_DIM)],
                    sem,
                ).start()

        def drain(q, sem):
            for k in range(GSZ):
                pltpu.make_async_copy(
                    tabT_hbm.at[:, pl.ds(0, 128)],
                    tc_v.at[pl.ds((q * GSZ + k) * BIGRAM_DIM, BIGRAM_DIM)],
                    sem,
                ).wait()

        def extract(lvec, tpos0, s, q):
            for k in range(GSZ):
                lane = lvec[GSZ * s + k]
                tpos = tpos0 + jnp.int32(GSZ * s + k)
                row0 = (q * GSZ + k) * BIGRAM_DIM
                for k2 in range(BIGRAM_DIM // LANES):
                    vec = plsc.load_gather(
                        tc_v,
                        [_splat(row0 + k2 * LANES) + _iota16(), _splat(lane)],
                    )
                    plsc.store_scatter(
                        col_v, [_splat(tpos), _splat(k2 * LANES) + _iota16()],
                        vec,
                    )

        def flush(blk):
            off = pl.multiple_of(base + blk * jnp.int32(128), 128)
            pltpu.sync_copy(col_v, outT_hbm.at[pl.ds(off, 128)])

        def group_vecs(g):
            vec = idx_v[pl.ds(g * LANES, LANES)]
            return (lax.shift_right_logical(vec, 7), vec & jnp.int32(127),
                    (g & jnp.int32(7)) * LANES)

        cv0, lv0, tp0 = group_vecs(jnp.int32(0))
        fire(cv0, 0, 0, sems[0])
        fire(cv0, 1, 1, sems[1])

        def pipe(i, carry):
            gv = [group_vecs(jnp.int32(3) * i + jnp.int32(m))
                  for m in range(4)]
            for j in range(12):
                sr_f = 2 + j
                fire(gv[sr_f // 4][0], sr_f % 4, sr_f % 3, sems[sr_f % 3])
                sr_d = j
                drain(sr_d % 3, sems[sr_d % 3])
                extract(gv[sr_d // 4][1], gv[sr_d // 4][2], sr_d % 4,
                        sr_d % 3)

                @pl.when(((jnp.int32(12) * i + jnp.int32(sr_d))
                          & jnp.int32(31)) == jnp.int32(31))
                def _():
                    flush(lax.shift_right_logical(
                        jnp.int32(12) * i + jnp.int32(sr_d), 5))

            return carry

        lax.fori_loop(0, NPIPE, pipe, 0)

        gvs = {}

        def gv_of(sr):
            g = sr // 4
            if g not in gvs:
                gvs[g] = group_vecs(jnp.int32(g))
            return gvs[g]

        for t in range(12 * NPIPE, NSR):
            if t + 2 < NSR:
                fire(gv_of(t + 2)[0], (t + 2) % 4, (t + 2) % 3,
                     sems[(t + 2) % 3])
            drain(t % 3, sems[t % 3])
            extract(gv_of(t)[1], gv_of(t)[2], t % 4, t % 3)
            if (t & 31) == 31:
                flush(jnp.int32(t >> 5))

    return _sc_body


def _sc_hash_gather(tok_padded, tableT, chunk):
    mesh = plsc.VectorSubcoreMesh(
        core_axis_name="c", subcore_axis_name="s", num_cores=2, num_subcores=16
    )
    return pl.kernel(
        _make_sc_body(chunk * CHUNK),
        out_type=jax.ShapeDtypeStruct((CHUNK, BIGRAM_DIM), jnp.float32),
        mesh=mesh,
        scratch_types=[
            pltpu.VMEM((PAD + PER_W,), jnp.int32),
            pltpu.VMEM((PER_W,), jnp.int32),
            pltpu.VMEM((3 * GSZ * BIGRAM_DIM, 128), jnp.float32),
            pltpu.VMEM((128, BIGRAM_DIM), jnp.float32),
            pltpu.SemaphoreType.DMA,
            pltpu.SemaphoreType.DMA,
            pltpu.SemaphoreType.DMA,
        ],
        compiler_params=pltpu.CompilerParams(needs_layout_passes=False),
    )(tok_padded, tableT)


def _tc_proj_body(scale_ref, g_ref, p_ref, o_ref):
    acc = lax.dot_general(
        g_ref[...], p_ref[...], (((1,), (1,)), ((), ())),
        preferred_element_type=jnp.float32,
    )
    o_ref[...] = acc * scale_ref[0, 0]


def _tc_proj(gathered, proj, scale):
    return pl.pallas_call(
        _tc_proj_body,
        grid=(TOKENS // TOK_BLK,),
        in_specs=[
            pl.BlockSpec(memory_space=pltpu.SMEM),
            pl.BlockSpec((TOK_BLK, BIGRAM_DIM), lambda i: (i, 0)),
            pl.BlockSpec((MODEL_DIM, BIGRAM_DIM), lambda i: (0, 0)),
        ],
        out_specs=pl.BlockSpec((TOK_BLK, MODEL_DIM), lambda i: (i, 0)),
        out_shape=jax.ShapeDtypeStruct((TOKENS, MODEL_DIM), jnp.float32),
    )(scale.reshape(1, 1), gathered, proj)


def kernel(tokens, embed_weight, proj_weight, scale):
    tok_flat = tokens.astype(jnp.int32).reshape(-1)
    tok_padded = jnp.concatenate([jnp.zeros((PAD,), jnp.int32), tok_flat])
    gathered = _sc_hash_gather(tok_padded, embed_weight.T, 0)
    out = _tc_proj(gathered, proj_weight, scale)
    return out.reshape(BATCH, SEQ, MODEL_DIM)
